# Initial kernel scaffold; baseline (speedup 1.0000x reference)
import jax
import jax.numpy as jnp
from jax.experimental import pallas as pl


def kernel(x_src, x_dst, edge_index, W, b):
    raise NotImplementedError("write your pallas kernel here")


# speedup vs baseline: 3.3258x; 3.3258x over previous
import dataclasses
import functools

import jax
import jax.numpy as jnp
from jax import lax
from jax.experimental import pallas as pl
from jax.experimental.pallas import tpu as pltpu
from jax.experimental.pallas import tpu_sc as plsc

N_TILES = 16
N_CORES = 2
N_WORK = N_CORES * N_TILES
DH = 128
LANES = 16
B = 128
NBC = 16


def _sc_aggregate(x_lo, x_hi, src3, dst3, n_pad):
    _, NB, _ = src3.shape
    NCH = NB // NBC
    RPT = n_pad // N_TILES
    CROWS = n_pad // 128
    mesh = plsc.VectorSubcoreMesh(core_axis_name="c", subcore_axis_name="s")
    cp = pltpu.CompilerParams()
    if "needs_layout_passes" in pltpu.CompilerParams.__dataclass_fields__:
        cp = dataclasses.replace(cp, needs_layout_passes=False)

    @functools.partial(
        pl.kernel,
        compiler_params=cp,
        out_type=(
            jax.ShapeDtypeStruct((n_pad, DH), jnp.float32),
            jax.ShapeDtypeStruct((n_pad, DH), jnp.float32),
            jax.ShapeDtypeStruct((N_WORK, CROWS, 128), jnp.float32),
        ),
        mesh=mesh,
        scratch_types=[
            pltpu.VMEM((NBC, B), jnp.int32),
            pltpu.VMEM((NBC, B), jnp.int32),
            pltpu.VMEM((B, DH), jnp.float32),
            pltpu.VMEM((CROWS, 128), jnp.float32),
            pltpu.VMEM_SHARED((n_pad, DH), jnp.float32),
        ],
    )
    def agg(xlo_hbm, xhi_hbm, src_hbm, dst_hbm,
            slo_hbm, shi_hbm, cnt_hbm,
            srcv, dstv, rows, cntv, acc):
        c = lax.axis_index("c")
        s = lax.axis_index("s")
        zero16 = jnp.zeros((LANES,), jnp.float32)
        one16 = jnp.ones((LANES,), jnp.float32)

        @pl.loop(0, B)
        def _(i):
            @pl.loop(0, DH // LANES)
            def _(j):
                rows[i, pl.ds(j * LANES, LANES)] = zero16

        @pl.loop(0, CROWS)
        def _(i):
            @pl.loop(0, 128 // LANES)
            def _(j):
                cntv[i, pl.ds(j * LANES, LANES)] = zero16

        @pl.loop(0, RPT // B)
        def _(k):
            pltpu.sync_copy(rows, acc.at[pl.ds(s * RPT + k * B, B)])

        plsc.subcore_barrier()

        @pl.loop(0, NCH)
        def _(ch):
            pltpu.sync_copy(src_hbm.at[s].at[pl.ds(ch * NBC, NBC)], srcv)
            pltpu.sync_copy(dst_hbm.at[s].at[pl.ds(ch * NBC, NBC)], dstv)

            @pl.loop(0, NBC)
            def _(j):
                @pl.when(c == 0)
                def _():
                    pltpu.sync_copy(xlo_hbm.at[srcv.at[j]], rows)

                @pl.when(c == 1)
                def _():
                    pltpu.sync_copy(xhi_hbm.at[srcv.at[j]], rows)

                pltpu.sync_copy(rows, acc.at[dstv.at[j]], add=True)

                @pl.when(lax.rem(j, 2) == c)
                def _():
                    @pl.loop(0, B // LANES)
                    def _(k):
                        d = dstv[j, pl.ds(k * LANES, LANES)]
                        plsc.addupdate_scatter(
                            cntv,
                            [lax.shift_right_logical(d, 7),
                             lax.bitwise_and(d, 127)],
                            one16)

        plsc.subcore_barrier()

        stripe = pl.ds(s * RPT, RPT)

        @pl.when(c == 0)
        def _():
            pltpu.sync_copy(acc.at[stripe], slo_hbm.at[stripe])

        @pl.when(c == 1)
        def _():
            pltpu.sync_copy(acc.at[stripe], shi_hbm.at[stripe])

        pltpu.sync_copy(cntv, cnt_hbm.at[c * N_TILES + s])

    return agg(x_lo, x_hi, src3, dst3)


def _tc_head(x_dst, sum_lo, sum_hi, cnt_t, Wt, b2):
    n, d = x_dst.shape
    out = Wt.shape[1]
    BLK = 1000

    def body(xd, slo, shi, ct, wt, bb, o):
        cnt = jnp.sum(ct[...], axis=1, keepdims=True)
        scale = 1.0 / jnp.maximum(cnt, 1.0)
        acc = jnp.dot(xd[...], wt[0:d, :], precision=lax.Precision.HIGHEST,
                      preferred_element_type=jnp.float32)
        acc = acc + jnp.dot(slo[...] * scale, wt[d:d + DH, :],
                            precision=lax.Precision.HIGHEST,
                            preferred_element_type=jnp.float32)
        acc = acc + jnp.dot(shi[...] * scale, wt[d + DH:d + 2 * DH, :],
                            precision=lax.Precision.HIGHEST,
                            preferred_element_type=jnp.float32)
        o[...] = jnp.maximum(acc + bb[...], 0.0)

    return pl.pallas_call(
        body,
        grid=(n // BLK,),
        in_specs=[
            pl.BlockSpec((BLK, d), lambda i: (i, 0)),
            pl.BlockSpec((BLK, DH), lambda i: (i, 0)),
            pl.BlockSpec((BLK, DH), lambda i: (i, 0)),
            pl.BlockSpec((BLK, N_WORK), lambda i: (i, 0)),
            pl.BlockSpec((d + 2 * DH, out), lambda i: (0, 0)),
            pl.BlockSpec((1, out), lambda i: (0, 0)),
        ],
        out_specs=pl.BlockSpec((BLK, out), lambda i: (i, 0)),
        out_shape=jax.ShapeDtypeStruct((n, out), jnp.float32),
    )(x_dst, sum_lo, sum_hi, cnt_t, Wt, b2)


def kernel(x_src, x_dst, edge_index, W, b):
    n_dst = x_dst.shape[0]
    n_pad = ((n_dst + 128 * N_TILES - 1) // (128 * N_TILES)) * (128 * N_TILES)
    src = edge_index[0].astype(jnp.int32)
    dst = edge_index[1].astype(jnp.int32)
    e = src.shape[0]
    epw = B * NBC
    e_pad = ((e + N_TILES * epw - 1) // (N_TILES * epw)) * (N_TILES * epw)
    src = jnp.concatenate([src, jnp.zeros((e_pad - e,), jnp.int32)])
    dst = jnp.concatenate(
        [dst, jnp.full((e_pad - e,), n_pad - 1, jnp.int32)])
    ept = e_pad // N_TILES
    nb = ept // B
    src3 = src.reshape(N_TILES, nb, B)
    dst3 = dst.reshape(N_TILES, nb, B)
    x_lo = x_src[:, :DH]
    x_hi = x_src[:, DH:]
    sum_lo, sum_hi, cnt = _sc_aggregate(x_lo, x_hi, src3, dst3, n_pad)
    cnt_t = cnt.reshape(N_WORK, n_pad).T
    return _tc_head(x_dst, sum_lo[:n_dst], sum_hi[:n_dst], cnt_t[:n_dst],
                    W.T, b.reshape(1, -1))


# speedup vs baseline: 3.9631x; 1.1916x over previous
import dataclasses
import functools

import jax
import jax.numpy as jnp
from jax import lax
from jax.experimental import pallas as pl
from jax.experimental.pallas import tpu as pltpu
from jax.experimental.pallas import tpu_sc as plsc

N_TILES = 16
N_CORES = 2
N_WORK = N_CORES * N_TILES
DH = 128
LANES = 16
B = 128
NBC = 8


def _sc_aggregate(x_lo, x_hi, src3, dst3, n_pad):
    _, NB, _ = src3.shape
    NCH = NB // NBC
    RPT = n_pad // N_TILES
    CROWS = n_pad // 128
    mesh = plsc.VectorSubcoreMesh(core_axis_name="c", subcore_axis_name="s")
    cp = pltpu.CompilerParams()
    if "needs_layout_passes" in pltpu.CompilerParams.__dataclass_fields__:
        cp = dataclasses.replace(cp, needs_layout_passes=False)

    @functools.partial(
        pl.kernel,
        compiler_params=cp,
        out_type=(
            jax.ShapeDtypeStruct((n_pad, DH), jnp.float32),
            jax.ShapeDtypeStruct((n_pad, DH), jnp.float32),
            jax.ShapeDtypeStruct((N_WORK, CROWS, 128), jnp.float32),
        ),
        mesh=mesh,
        scratch_types=[
            pltpu.VMEM((2, NBC, B), jnp.int32),
            pltpu.VMEM((2, NBC, B), jnp.int32),
            pltpu.VMEM((2, B, DH), jnp.float32),
            pltpu.VMEM((CROWS, 128), jnp.float32),
            pltpu.VMEM_SHARED((n_pad, DH), jnp.float32),
            pltpu.SemaphoreType.DMA,
            pltpu.SemaphoreType.DMA,
        ],
    )
    def agg(xlo_hbm, xhi_hbm, src_hbm, dst_hbm,
            slo_hbm, shi_hbm, cnt_hbm,
            srcv, dstv, rows, cntv, acc, g0, g1):
        c = lax.axis_index("c")
        s = lax.axis_index("s")
        zero16 = jnp.zeros((LANES,), jnp.float32)
        one16 = jnp.ones((LANES,), jnp.float32)

        @pl.loop(0, B)
        def _(i):
            @pl.loop(0, DH // LANES)
            def _(j):
                rows[0, i, pl.ds(j * LANES, LANES)] = zero16

        @pl.loop(0, CROWS)
        def _(i):
            @pl.loop(0, 128 // LANES)
            def _(j):
                cntv[i, pl.ds(j * LANES, LANES)] = zero16

        @pl.loop(0, RPT // B)
        def _(k):
            pltpu.sync_copy(rows.at[0], acc.at[pl.ds(s * RPT + k * B, B)])

        plsc.subcore_barrier()

        def stage(ch):
            buf = lax.rem(ch, 2)
            pltpu.sync_copy(src_hbm.at[s].at[pl.ds(ch * NBC, NBC)],
                            srcv.at[buf])
            pltpu.sync_copy(dst_hbm.at[s].at[pl.ds(ch * NBC, NBC)],
                            dstv.at[buf])

        def src_row(j):
            return srcv.at[lax.rem(j // NBC, 2)].at[lax.rem(j, NBC)]

        def dst_row(j):
            return dstv.at[lax.rem(j // NBC, 2)].at[lax.rem(j, NBC)]

        def start_gather(j, rbuf, sem):
            @pl.when(c == 0)
            def _():
                pltpu.async_copy(xlo_hbm.at[src_row(j)], rows.at[rbuf], sem)

            @pl.when(c == 1)
            def _():
                pltpu.async_copy(xhi_hbm.at[src_row(j)], rows.at[rbuf], sem)

        def wait_gather(rbuf, sem):
            pltpu.make_async_copy(xlo_hbm.at[pl.ds(0, B)], rows.at[rbuf],
                                  sem).wait()

        def consume(j, rbuf):
            pltpu.sync_copy(rows.at[rbuf], acc.at[dst_row(j)], add=True)
            @pl.when(lax.rem(j, 2) == c)
            def _():
                @pl.loop(0, B // LANES)
                def _(k):
                    d = dstv[lax.rem(j // NBC, 2), lax.rem(j, NBC),
                             pl.ds(k * LANES, LANES)]
                    plsc.addupdate_scatter(
                        cntv,
                        [lax.shift_right_logical(d, 7),
                         lax.bitwise_and(d, 127)],
                        one16)

        NB_TOT = NCH * NBC
        stage(0)
        start_gather(0, 0, g0)

        @pl.loop(0, NB_TOT // 2)
        def _(p):
            j0 = 2 * p
            j1 = j0 + 1
            j2 = j0 + 2

            @pl.when(lax.rem(j1, NBC) == 0)
            def _():
                stage(j1 // NBC)
            start_gather(j1, 1, g1)
            wait_gather(0, g0)
            consume(j0, 0)

            @pl.when(j2 < NB_TOT)
            def _():
                @pl.when(lax.rem(j2, NBC) == 0)
                def _():
                    stage(j2 // NBC)
                start_gather(j2, 0, g0)
            wait_gather(1, g1)
            consume(j1, 1)

        plsc.subcore_barrier()

        stripe = pl.ds(s * RPT, RPT)

        @pl.when(c == 0)
        def _():
            pltpu.sync_copy(acc.at[stripe], slo_hbm.at[stripe])

        @pl.when(c == 1)
        def _():
            pltpu.sync_copy(acc.at[stripe], shi_hbm.at[stripe])

        pltpu.sync_copy(cntv, cnt_hbm.at[c * N_TILES + s])

    return agg(x_lo, x_hi, src3, dst3)


def _tc_head(x_dst, sum_lo, sum_hi, cnt_t, Wt, b2):
    n, d = x_dst.shape
    out = Wt.shape[1]
    BLK = 1000

    def body(xd, slo, shi, ct, wt, bb, o):
        cnt = jnp.sum(ct[...], axis=1, keepdims=True)
        scale = 1.0 / jnp.maximum(cnt, 1.0)
        acc = jnp.dot(xd[...], wt[0:d, :], precision=lax.Precision.HIGHEST,
                      preferred_element_type=jnp.float32)
        acc = acc + jnp.dot(slo[...] * scale, wt[d:d + DH, :],
                            precision=lax.Precision.HIGHEST,
                            preferred_element_type=jnp.float32)
        acc = acc + jnp.dot(shi[...] * scale, wt[d + DH:d + 2 * DH, :],
                            precision=lax.Precision.HIGHEST,
                            preferred_element_type=jnp.float32)
        o[...] = jnp.maximum(acc + bb[...], 0.0)

    return pl.pallas_call(
        body,
        grid=(n // BLK,),
        in_specs=[
            pl.BlockSpec((BLK, d), lambda i: (i, 0)),
            pl.BlockSpec((BLK, DH), lambda i: (i, 0)),
            pl.BlockSpec((BLK, DH), lambda i: (i, 0)),
            pl.BlockSpec((BLK, N_WORK), lambda i: (i, 0)),
            pl.BlockSpec((d + 2 * DH, out), lambda i: (0, 0)),
            pl.BlockSpec((1, out), lambda i: (0, 0)),
        ],
        out_specs=pl.BlockSpec((BLK, out), lambda i: (i, 0)),
        out_shape=jax.ShapeDtypeStruct((n, out), jnp.float32),
    )(x_dst, sum_lo, sum_hi, cnt_t, Wt, b2)


def kernel(x_src, x_dst, edge_index, W, b):
    n_dst = x_dst.shape[0]
    n_pad = ((n_dst + 128 * N_TILES - 1) // (128 * N_TILES)) * (128 * N_TILES)
    src = edge_index[0].astype(jnp.int32)
    dst = edge_index[1].astype(jnp.int32)
    e = src.shape[0]
    epw = B * NBC
    e_pad = ((e + N_TILES * epw - 1) // (N_TILES * epw)) * (N_TILES * epw)
    src = jnp.concatenate([src, jnp.zeros((e_pad - e,), jnp.int32)])
    dst = jnp.concatenate(
        [dst, jnp.full((e_pad - e,), n_pad - 1, jnp.int32)])
    ept = e_pad // N_TILES
    nb = ept // B
    src3 = src.reshape(N_TILES, nb, B)
    dst3 = dst.reshape(N_TILES, nb, B)
    x_lo = x_src[:, :DH]
    x_hi = x_src[:, DH:]
    sum_lo, sum_hi, cnt = _sc_aggregate(x_lo, x_hi, src3, dst3, n_pad)
    cnt_t = cnt.reshape(N_WORK, n_pad).T
    return _tc_head(x_dst, sum_lo[:n_dst], sum_hi[:n_dst], cnt_t[:n_dst],
                    W.T, b.reshape(1, -1))


# speedup vs baseline: 4.0100x; 1.0118x over previous
import dataclasses
import functools

import jax
import jax.numpy as jnp
from jax import lax
from jax.experimental import pallas as pl
from jax.experimental.pallas import tpu as pltpu
from jax.experimental.pallas import tpu_sc as plsc

N_TILES = 16
N_CORES = 2
N_WORK = N_CORES * N_TILES
DH = 128
LANES = 16
B = 64
NBC = 8


def _sc_aggregate(x_lo, x_hi, src3, dst3, n_pad):
    _, NB, _ = src3.shape
    NCH = NB // NBC
    RPT = n_pad // N_TILES
    CROWS = n_pad // 128
    mesh = plsc.VectorSubcoreMesh(core_axis_name="c", subcore_axis_name="s")
    cp = pltpu.CompilerParams()
    if "needs_layout_passes" in pltpu.CompilerParams.__dataclass_fields__:
        cp = dataclasses.replace(cp, needs_layout_passes=False)

    @functools.partial(
        pl.kernel,
        compiler_params=cp,
        out_type=(
            jax.ShapeDtypeStruct((n_pad, DH), jnp.float32),
            jax.ShapeDtypeStruct((n_pad, DH), jnp.float32),
            jax.ShapeDtypeStruct((N_WORK, CROWS, 128), jnp.float32),
        ),
        mesh=mesh,
        scratch_types=[
            pltpu.VMEM((2, NBC, B), jnp.int32),
            pltpu.VMEM((2, NBC, B), jnp.int32),
            pltpu.VMEM((2, B, DH), jnp.float32),
            pltpu.VMEM((CROWS, 128), jnp.float32),
            pltpu.VMEM_SHARED((n_pad, DH), jnp.float32),
            pltpu.SemaphoreType.DMA,
            pltpu.SemaphoreType.DMA,
        ],
    )
    def agg(xlo_hbm, xhi_hbm, src_hbm, dst_hbm,
            slo_hbm, shi_hbm, cnt_hbm,
            srcv, dstv, rows, cntv, acc, g0, g1):
        c = lax.axis_index("c")
        s = lax.axis_index("s")
        zero16 = jnp.zeros((LANES,), jnp.float32)
        one16 = jnp.ones((LANES,), jnp.float32)

        @pl.loop(0, B)
        def _(i):
            @pl.loop(0, DH // LANES)
            def _(j):
                rows[0, i, pl.ds(j * LANES, LANES)] = zero16

        @pl.loop(0, CROWS)
        def _(i):
            @pl.loop(0, 128 // LANES)
            def _(j):
                cntv[i, pl.ds(j * LANES, LANES)] = zero16

        @pl.loop(0, RPT // B)
        def _(k):
            pltpu.sync_copy(rows.at[0], acc.at[pl.ds(s * RPT + k * B, B)])

        plsc.subcore_barrier()

        def stage(ch):
            buf = lax.rem(ch, 2)
            pltpu.sync_copy(src_hbm.at[s].at[pl.ds(ch * NBC, NBC)],
                            srcv.at[buf])
            pltpu.sync_copy(dst_hbm.at[s].at[pl.ds(ch * NBC, NBC)],
                            dstv.at[buf])

        def src_row(j):
            return srcv.at[lax.rem(j // NBC, 2)].at[lax.rem(j, NBC)]

        def dst_row(j):
            return dstv.at[lax.rem(j // NBC, 2)].at[lax.rem(j, NBC)]

        def start_gather(j, rbuf, sem):
            @pl.when(c == 0)
            def _():
                pltpu.async_copy(xlo_hbm.at[src_row(j)], rows.at[rbuf], sem)

            @pl.when(c == 1)
            def _():
                pltpu.async_copy(xhi_hbm.at[src_row(j)], rows.at[rbuf], sem)

        def wait_gather(rbuf, sem):
            pltpu.make_async_copy(xlo_hbm.at[pl.ds(0, B)], rows.at[rbuf],
                                  sem).wait()

        def consume(j, rbuf):
            pltpu.sync_copy(rows.at[rbuf], acc.at[dst_row(j)], add=True)
            @pl.when(lax.rem(j, 2) == c)
            def _():
                @pl.loop(0, B // LANES)
                def _(k):
                    d = dstv[lax.rem(j // NBC, 2), lax.rem(j, NBC),
                             pl.ds(k * LANES, LANES)]
                    plsc.addupdate_scatter(
                        cntv,
                        [lax.shift_right_logical(d, 7),
                         lax.bitwise_and(d, 127)],
                        one16)

        NB_TOT = NCH * NBC
        stage(0)
        start_gather(0, 0, g0)

        @pl.loop(0, NB_TOT // 2)
        def _(p):
            j0 = 2 * p
            j1 = j0 + 1
            j2 = j0 + 2

            @pl.when(lax.rem(j1, NBC) == 0)
            def _():
                stage(j1 // NBC)
            start_gather(j1, 1, g1)
            wait_gather(0, g0)
            consume(j0, 0)

            @pl.when(j2 < NB_TOT)
            def _():
                @pl.when(lax.rem(j2, NBC) == 0)
                def _():
                    stage(j2 // NBC)
                start_gather(j2, 0, g0)
            wait_gather(1, g1)
            consume(j1, 1)

        plsc.subcore_barrier()

        stripe = pl.ds(s * RPT, RPT)

        @pl.when(c == 0)
        def _():
            pltpu.sync_copy(acc.at[stripe], slo_hbm.at[stripe])

        @pl.when(c == 1)
        def _():
            pltpu.sync_copy(acc.at[stripe], shi_hbm.at[stripe])

        pltpu.sync_copy(cntv, cnt_hbm.at[c * N_TILES + s])

    return agg(x_lo, x_hi, src3, dst3)


def _tc_head(x_dst, sum_lo, sum_hi, cnt_t, Wt, b2):
    n, d = x_dst.shape
    out = Wt.shape[1]
    BLK = 1000

    def body(xd, slo, shi, ct, wt, bb, o):
        cnt = jnp.sum(ct[...], axis=1, keepdims=True)
        scale = 1.0 / jnp.maximum(cnt, 1.0)
        acc = jnp.dot(xd[...], wt[0:d, :], precision=lax.Precision.HIGHEST,
                      preferred_element_type=jnp.float32)
        acc = acc + jnp.dot(slo[...] * scale, wt[d:d + DH, :],
                            precision=lax.Precision.HIGHEST,
                            preferred_element_type=jnp.float32)
        acc = acc + jnp.dot(shi[...] * scale, wt[d + DH:d + 2 * DH, :],
                            precision=lax.Precision.HIGHEST,
                            preferred_element_type=jnp.float32)
        o[...] = jnp.maximum(acc + bb[...], 0.0)

    return pl.pallas_call(
        body,
        grid=(n // BLK,),
        in_specs=[
            pl.BlockSpec((BLK, d), lambda i: (i, 0)),
            pl.BlockSpec((BLK, DH), lambda i: (i, 0)),
            pl.BlockSpec((BLK, DH), lambda i: (i, 0)),
            pl.BlockSpec((BLK, N_WORK), lambda i: (i, 0)),
            pl.BlockSpec((d + 2 * DH, out), lambda i: (0, 0)),
            pl.BlockSpec((1, out), lambda i: (0, 0)),
        ],
        out_specs=pl.BlockSpec((BLK, out), lambda i: (i, 0)),
        out_shape=jax.ShapeDtypeStruct((n, out), jnp.float32),
    )(x_dst, sum_lo, sum_hi, cnt_t, Wt, b2)


def kernel(x_src, x_dst, edge_index, W, b):
    n_dst = x_dst.shape[0]
    n_pad = ((n_dst + 128 * N_TILES - 1) // (128 * N_TILES)) * (128 * N_TILES)
    src = edge_index[0].astype(jnp.int32)
    dst = edge_index[1].astype(jnp.int32)
    e = src.shape[0]
    epw = B * NBC
    e_pad = ((e + N_TILES * epw - 1) // (N_TILES * epw)) * (N_TILES * epw)
    src = jnp.concatenate([src, jnp.zeros((e_pad - e,), jnp.int32)])
    dst = jnp.concatenate(
        [dst, jnp.full((e_pad - e,), n_pad - 1, jnp.int32)])
    ept = e_pad // N_TILES
    nb = ept // B
    src3 = src.reshape(N_TILES, nb, B)
    dst3 = dst.reshape(N_TILES, nb, B)
    x_lo = x_src[:, :DH]
    x_hi = x_src[:, DH:]
    sum_lo, sum_hi, cnt = _sc_aggregate(x_lo, x_hi, src3, dst3, n_pad)
    cnt_t = cnt.reshape(N_WORK, n_pad).T
    return _tc_head(x_dst, sum_lo[:n_dst], sum_hi[:n_dst], cnt_t[:n_dst],
                    W.T, b.reshape(1, -1))


# speedup vs baseline: 4.0374x; 1.0068x over previous
import dataclasses
import functools

import jax
import jax.numpy as jnp
from jax import lax
from jax.experimental import pallas as pl
from jax.experimental.pallas import tpu as pltpu
from jax.experimental.pallas import tpu_sc as plsc

N_TILES = 16
N_CORES = 2
N_WORK = N_CORES * N_TILES
DH = 128
LANES = 16
B = 64
NBC = 8


def _sc_aggregate(x_lo, x_hi, src3, dst3, n_pad):
    _, NB, _ = src3.shape
    NCH = NB // NBC
    RPT = n_pad // N_TILES
    CROWS = n_pad // 128
    mesh = plsc.VectorSubcoreMesh(core_axis_name="c", subcore_axis_name="s")
    cp = pltpu.CompilerParams()
    if "needs_layout_passes" in pltpu.CompilerParams.__dataclass_fields__:
        cp = dataclasses.replace(cp, needs_layout_passes=False)

    @functools.partial(
        pl.kernel,
        compiler_params=cp,
        out_type=(
            jax.ShapeDtypeStruct((n_pad, DH), jnp.float32),
            jax.ShapeDtypeStruct((n_pad, DH), jnp.float32),
            jax.ShapeDtypeStruct((N_WORK, CROWS, 128), jnp.float32),
        ),
        mesh=mesh,
        scratch_types=[
            pltpu.VMEM((2, NBC, B), jnp.int32),
            pltpu.VMEM((2, NBC, B), jnp.int32),
            pltpu.VMEM((2, B, DH), jnp.float32),
            pltpu.VMEM((CROWS, 128), jnp.float32),
            pltpu.VMEM_SHARED((n_pad, DH), jnp.float32),
            pltpu.SemaphoreType.DMA,
            pltpu.SemaphoreType.DMA,
        ],
    )
    def agg(xlo_hbm, xhi_hbm, src_hbm, dst_hbm,
            slo_hbm, shi_hbm, cnt_hbm,
            srcv, dstv, rows, cntv, acc, g0, g1):
        c = lax.axis_index("c")
        s = lax.axis_index("s")
        zero16 = jnp.zeros((LANES,), jnp.float32)
        one16 = jnp.ones((LANES,), jnp.float32)

        @pl.loop(0, B)
        def _(i):
            @pl.loop(0, DH // LANES)
            def _(j):
                rows[0, i, pl.ds(j * LANES, LANES)] = zero16

        @pl.loop(0, CROWS)
        def _(i):
            @pl.loop(0, 128 // LANES)
            def _(j):
                cntv[i, pl.ds(j * LANES, LANES)] = zero16

        @pl.loop(0, RPT // B)
        def _(k):
            pltpu.sync_copy(rows.at[0], acc.at[pl.ds(s * RPT + k * B, B)])

        plsc.subcore_barrier()

        def stage(ch):
            buf = lax.rem(ch, 2)
            pltpu.sync_copy(src_hbm.at[s].at[pl.ds(ch * NBC, NBC)],
                            srcv.at[buf])
            pltpu.sync_copy(dst_hbm.at[s].at[pl.ds(ch * NBC, NBC)],
                            dstv.at[buf])

        def src_row(j):
            return srcv.at[lax.rem(j // NBC, 2)].at[lax.rem(j, NBC)]

        def dst_row(j):
            return dstv.at[lax.rem(j // NBC, 2)].at[lax.rem(j, NBC)]

        def start_gather(j, rbuf, sem):
            @pl.when(c == 0)
            def _():
                pltpu.async_copy(xlo_hbm.at[src_row(j)], rows.at[rbuf], sem)

            @pl.when(c == 1)
            def _():
                pltpu.async_copy(xhi_hbm.at[src_row(j)], rows.at[rbuf], sem)

        def wait_gather(rbuf, sem):
            pltpu.make_async_copy(xlo_hbm.at[pl.ds(0, B)], rows.at[rbuf],
                                  sem).wait()

        def consume(j, rbuf):
            pltpu.sync_copy(rows.at[rbuf], acc.at[dst_row(j)], add=True)
            @pl.when(lax.rem(j, 2) == c)
            def _():
                @pl.loop(0, B // LANES)
                def _(k):
                    d = dstv[lax.rem(j // NBC, 2), lax.rem(j, NBC),
                             pl.ds(k * LANES, LANES)]
                    plsc.addupdate_scatter(
                        cntv,
                        [lax.shift_right_logical(d, 7),
                         lax.bitwise_and(d, 127)],
                        one16)

        NB_TOT = NCH * NBC
        stage(0)
        start_gather(0, 0, g0)

        @pl.loop(0, NB_TOT // 2)
        def _(p):
            j0 = 2 * p
            j1 = j0 + 1
            j2 = j0 + 2

            @pl.when(lax.rem(j1, NBC) == 0)
            def _():
                stage(j1 // NBC)
            start_gather(j1, 1, g1)
            wait_gather(0, g0)
            consume(j0, 0)

            @pl.when(j2 < NB_TOT)
            def _():
                @pl.when(lax.rem(j2, NBC) == 0)
                def _():
                    stage(j2 // NBC)
                start_gather(j2, 0, g0)
            wait_gather(1, g1)
            consume(j1, 1)

        plsc.subcore_barrier()

        stripe = pl.ds(s * RPT, RPT)

        @pl.when(c == 0)
        def _():
            pltpu.sync_copy(acc.at[stripe], slo_hbm.at[stripe])

        @pl.when(c == 1)
        def _():
            pltpu.sync_copy(acc.at[stripe], shi_hbm.at[stripe])

        pltpu.sync_copy(cntv, cnt_hbm.at[c * N_TILES + s])

    return agg(x_lo, x_hi, src3, dst3)


def _tc_self(x_dst, W1t, b2):
    n, d = x_dst.shape
    out = W1t.shape[1]
    BLK = 1000

    def body(xd, wt, bb, o):
        o[...] = jnp.dot(xd[...], wt[...], precision=lax.Precision.HIGHEST,
                         preferred_element_type=jnp.float32) + bb[...]

    return pl.pallas_call(
        body,
        grid=(n // BLK,),
        in_specs=[
            pl.BlockSpec((BLK, d), lambda i: (i, 0)),
            pl.BlockSpec((d, out), lambda i: (0, 0)),
            pl.BlockSpec((1, out), lambda i: (0, 0)),
        ],
        out_specs=pl.BlockSpec((BLK, out), lambda i: (i, 0)),
        out_shape=jax.ShapeDtypeStruct((n, out), jnp.float32),
    )(x_dst, W1t, b2)


def _tc_head(y0, sum_lo, sum_hi, cnt_t, W2t):
    n, _ = y0.shape
    out = W2t.shape[1]
    BLK = 1000

    def body(y0r, slo, shi, ct, wt, o):
        cnt = jnp.sum(ct[...], axis=1, keepdims=True)
        scale = 1.0 / jnp.maximum(cnt, 1.0)
        acc = y0r[...]
        acc = acc + jnp.dot(slo[...] * scale, wt[0:DH, :],
                            precision=lax.Precision.HIGHEST,
                            preferred_element_type=jnp.float32)
        acc = acc + jnp.dot(shi[...] * scale, wt[DH:2 * DH, :],
                            precision=lax.Precision.HIGHEST,
                            preferred_element_type=jnp.float32)
        o[...] = jnp.maximum(acc, 0.0)

    return pl.pallas_call(
        body,
        grid=(n // BLK,),
        in_specs=[
            pl.BlockSpec((BLK, out), lambda i: (i, 0)),
            pl.BlockSpec((BLK, DH), lambda i: (i, 0)),
            pl.BlockSpec((BLK, DH), lambda i: (i, 0)),
            pl.BlockSpec((BLK, N_WORK), lambda i: (i, 0)),
            pl.BlockSpec((2 * DH, out), lambda i: (0, 0)),
        ],
        out_specs=pl.BlockSpec((BLK, out), lambda i: (i, 0)),
        out_shape=jax.ShapeDtypeStruct((n, out), jnp.float32),
    )(y0, sum_lo, sum_hi, cnt_t, W2t)


def kernel(x_src, x_dst, edge_index, W, b):
    n_dst = x_dst.shape[0]
    n_pad = ((n_dst + 128 * N_TILES - 1) // (128 * N_TILES)) * (128 * N_TILES)
    src = edge_index[0].astype(jnp.int32)
    dst = edge_index[1].astype(jnp.int32)
    e = src.shape[0]
    epw = B * NBC
    e_pad = ((e + N_TILES * epw - 1) // (N_TILES * epw)) * (N_TILES * epw)
    src = jnp.concatenate([src, jnp.zeros((e_pad - e,), jnp.int32)])
    dst = jnp.concatenate(
        [dst, jnp.full((e_pad - e,), n_pad - 1, jnp.int32)])
    ept = e_pad // N_TILES
    nb = ept // B
    src3 = src.reshape(N_TILES, nb, B)
    dst3 = dst.reshape(N_TILES, nb, B)
    x_lo = x_src[:, :DH]
    x_hi = x_src[:, DH:]
    sum_lo, sum_hi, cnt = _sc_aggregate(x_lo, x_hi, src3, dst3, n_pad)
    Wt = W.T
    d = x_src.shape[1]
    y0 = _tc_self(x_dst, Wt[:d], b.reshape(1, -1))
    cnt_t = cnt.reshape(N_WORK, n_pad).T
    return _tc_head(y0, sum_lo[:n_dst], sum_hi[:n_dst], cnt_t[:n_dst],
                    Wt[d:])
